# baseline (speedup 1.0000x reference)
import functools

import jax
import jax.numpy as jnp
from jax.experimental import pallas as pl
from jax.experimental.pallas import tpu as pltpu

_TOPK = 64


def _sae_body(x_ref, enc_ref, dec_ref, bpre_ref, benc_ref, zn_ref, xt_ref,
              acc_ref, *, topk):
    x = x_ref[...]
    xb = x - bpre_ref[...]
    logits = jax.lax.dot_general(
        xb, enc_ref[...], (((1,), (1,)), ((), ())),
        preferred_element_type=jnp.float32,
        precision=jax.lax.Precision.DEFAULT)
    z = jnp.maximum(logits + benc_ref[...], 0.0)

    topkf = jnp.float32(topk)
    acc = jnp.zeros((z.shape[0], 1), jnp.int32)
    cnta = jnp.full((z.shape[0], 1), jnp.float32(z.shape[1]))
    for b in range(30, 7, -1):
        cand = acc | (1 << b)
        candf = jax.lax.bitcast_convert_type(cand, jnp.float32)
        cnt = jnp.sum(jnp.where(z >= candf, 1.0, 0.0), axis=1, keepdims=True)
        take = cnt >= topkf
        acc = jnp.where(take, cand, acc)
        cnta = jnp.where(take, cnt, cnta)
    acc_ref[...] = acc

    @pl.when(jnp.any(cnta != topkf))
    def _refine():
        a = acc_ref[...]
        for b in range(7, -1, -1):
            cand = a | (1 << b)
            candf = jax.lax.bitcast_convert_type(cand, jnp.float32)
            cnt = jnp.sum(jnp.where(z >= candf, 1.0, 0.0),
                          axis=1, keepdims=True)
            a = jnp.where(cnt >= topkf, cand, a)
        acc_ref[...] = a

    thr = jax.lax.bitcast_convert_type(acc_ref[...], jnp.float32)

    zs = jnp.where(z >= thr, z, 0.0)
    zn_ref[...] = zs
    xt = jax.lax.dot_general(
        zs, dec_ref[...], (((1,), (1,)), ((), ())),
        preferred_element_type=jnp.float32,
        precision=jax.lax.Precision.DEFAULT)
    xt_ref[...] = xt + bpre_ref[...]


def kernel(zL, dictionary_enc, dictionary_dec, bias_pre, bias_enc):
    B, D, L, H = zL.shape
    M = dictionary_enc.shape[0]
    N = B * D * L
    R = 512 if N % 512 == 0 else N
    grid = N // R

    x = zL.reshape(N, H)
    enc_t = dictionary_enc
    dec_t = dictionary_dec
    bpre = bias_pre.reshape(1, H)
    benc = bias_enc.reshape(1, M)

    z_n, x_tgt = pl.pallas_call(
        functools.partial(_sae_body, topk=_TOPK),
        grid=(grid,),
        in_specs=[
            pl.BlockSpec((R, H), lambda i: (i, 0)),
            pl.BlockSpec((M, H), lambda i: (0, 0)),
            pl.BlockSpec((H, M), lambda i: (0, 0)),
            pl.BlockSpec((1, H), lambda i: (0, 0)),
            pl.BlockSpec((1, M), lambda i: (0, 0)),
        ],
        out_specs=[
            pl.BlockSpec((R, M), lambda i: (i, 0)),
            pl.BlockSpec((R, H), lambda i: (i, 0)),
        ],
        out_shape=[
            jax.ShapeDtypeStruct((N, M), jnp.float32),
            jax.ShapeDtypeStruct((N, H), jnp.float32),
        ],
        scratch_shapes=[pltpu.VMEM((R, 1), jnp.int32)],
    )(x, enc_t, dec_t, bpre, benc)

    return z_n.reshape(B, D, L, M), x_tgt.reshape(B, D, L, H)


# speedup vs baseline: 1.0432x; 1.0432x over previous
import functools

import jax
import jax.numpy as jnp
from jax.experimental import pallas as pl
from jax.experimental.pallas import tpu as pltpu

_TOPK = 64


def _sae_body(x_ref, enc_ref, dec_ref, bpre_ref, benc_ref, zn_ref, xt_ref,
              zbuf_ref, acc_ref, *, topk):
    i = pl.program_id(0)
    slot = jax.lax.rem(i, 2)

    x = x_ref[...]
    xb = x - bpre_ref[...]
    logits = jax.lax.dot_general(
        xb, enc_ref[...], (((1,), (1,)), ((), ())),
        preferred_element_type=jnp.float32,
        precision=jax.lax.Precision.DEFAULT)
    zbuf_ref[slot] = jnp.maximum(logits + benc_ref[...], 0.0)

    z = zbuf_ref[1 - slot]
    topkf = jnp.float32(topk)
    acc = jnp.zeros((z.shape[0], 1), jnp.int32)
    cnta = jnp.full((z.shape[0], 1), jnp.float32(z.shape[1]))
    for b in range(30, 5, -1):
        cand = acc | (1 << b)
        candf = jax.lax.bitcast_convert_type(cand, jnp.float32)
        cnt = jnp.sum(jnp.where(z >= candf, 1.0, 0.0), axis=1, keepdims=True)
        take = cnt >= topkf
        acc = jnp.where(take, cand, acc)
        cnta = jnp.where(take, cnt, cnta)
    acc_ref[...] = acc

    @pl.when(jnp.any(cnta != topkf))
    def _refine():
        a = acc_ref[...]
        for b in range(5, -1, -1):
            cand = a | (1 << b)
            candf = jax.lax.bitcast_convert_type(cand, jnp.float32)
            cnt = jnp.sum(jnp.where(z >= candf, 1.0, 0.0),
                          axis=1, keepdims=True)
            a = jnp.where(cnt >= topkf, cand, a)
        acc_ref[...] = a

    thr = jax.lax.bitcast_convert_type(acc_ref[...], jnp.float32)

    zs = jnp.where(z >= thr, z, 0.0)
    zn_ref[...] = zs
    xt = jax.lax.dot_general(
        zs, dec_ref[...], (((1,), (1,)), ((), ())),
        preferred_element_type=jnp.float32,
        precision=jax.lax.Precision.DEFAULT)
    xt_ref[...] = xt + bpre_ref[...]


def kernel(zL, dictionary_enc, dictionary_dec, bias_pre, bias_enc):
    B, D, L, H = zL.shape
    M = dictionary_enc.shape[0]
    N = B * D * L
    R = 256 if N % 256 == 0 else N
    tiles = N // R

    x = zL.reshape(N, H)
    bpre = bias_pre.reshape(1, H)
    benc = bias_enc.reshape(1, M)

    last = tiles - 1
    z_n, x_tgt = pl.pallas_call(
        functools.partial(_sae_body, topk=_TOPK),
        grid=(tiles + 1,),
        in_specs=[
            pl.BlockSpec((R, H), lambda i: (jnp.minimum(i, last), 0)),
            pl.BlockSpec((M, H), lambda i: (0, 0)),
            pl.BlockSpec((H, M), lambda i: (0, 0)),
            pl.BlockSpec((1, H), lambda i: (0, 0)),
            pl.BlockSpec((1, M), lambda i: (0, 0)),
        ],
        out_specs=[
            pl.BlockSpec((R, M), lambda i: (jnp.maximum(i - 1, 0), 0)),
            pl.BlockSpec((R, H), lambda i: (jnp.maximum(i - 1, 0), 0)),
        ],
        out_shape=[
            jax.ShapeDtypeStruct((N, M), jnp.float32),
            jax.ShapeDtypeStruct((N, H), jnp.float32),
        ],
        scratch_shapes=[
            pltpu.VMEM((2, R, M), jnp.float32),
            pltpu.VMEM((R, 1), jnp.int32),
        ],
    )(x, dictionary_enc, dictionary_dec, bpre, benc)

    return z_n.reshape(B, D, L, M), x_tgt.reshape(B, D, L, H)


# speedup vs baseline: 1.1092x; 1.0633x over previous
import functools

import jax
import jax.numpy as jnp
from jax.experimental import pallas as pl
from jax.experimental.pallas import tpu as pltpu

_TOPK = 64


def _sae_body(x_ref, enc_ref, dec_ref, bpre_ref, benc_ref, zn_ref, xt_ref,
              acc_ref, *, topk):
    x = x_ref[...]
    xb = x - bpre_ref[...]
    logits = jax.lax.dot_general(
        xb, enc_ref[...], (((1,), (1,)), ((), ())),
        preferred_element_type=jnp.float32,
        precision=jax.lax.Precision.DEFAULT)
    z = jnp.maximum(logits + benc_ref[...], 0.0)

    topkf = jnp.float32(topk)
    acc = jnp.zeros((z.shape[0], 1), jnp.int32)
    cnta = jnp.full((z.shape[0], 1), jnp.float32(z.shape[1]))
    for b in range(30, 5, -1):
        cand = acc | (1 << b)
        candf = jax.lax.bitcast_convert_type(cand, jnp.float32)
        cnt = jnp.sum(jnp.where(z >= candf, 1.0, 0.0), axis=1, keepdims=True)
        take = cnt >= topkf
        acc = jnp.where(take, cand, acc)
        cnta = jnp.where(take, cnt, cnta)
    acc_ref[...] = acc

    @pl.when(jnp.any(cnta != topkf))
    def _refine():
        a = acc_ref[...]
        for b in range(5, -1, -1):
            cand = a | (1 << b)
            candf = jax.lax.bitcast_convert_type(cand, jnp.float32)
            cnt = jnp.sum(jnp.where(z >= candf, 1.0, 0.0),
                          axis=1, keepdims=True)
            a = jnp.where(cnt >= topkf, cand, a)
        acc_ref[...] = a

    thr = jax.lax.bitcast_convert_type(acc_ref[...], jnp.float32)

    zs = jnp.where(z >= thr, z, 0.0)
    zn_ref[...] = zs
    xt = jax.lax.dot_general(
        zs, dec_ref[...], (((1,), (1,)), ((), ())),
        preferred_element_type=jnp.float32,
        precision=jax.lax.Precision.DEFAULT)
    xt_ref[...] = xt + bpre_ref[...]


def kernel(zL, dictionary_enc, dictionary_dec, bias_pre, bias_enc):
    B, D, L, H = zL.shape
    M = dictionary_enc.shape[0]
    N = B * D * L
    R = 512 if N % 512 == 0 else N
    grid = N // R

    x = zL.reshape(N, H)
    enc_t = dictionary_enc
    dec_t = dictionary_dec
    bpre = bias_pre.reshape(1, H)
    benc = bias_enc.reshape(1, M)

    z_n, x_tgt = pl.pallas_call(
        functools.partial(_sae_body, topk=_TOPK),
        grid=(grid,),
        in_specs=[
            pl.BlockSpec((R, H), lambda i: (i, 0)),
            pl.BlockSpec((M, H), lambda i: (0, 0)),
            pl.BlockSpec((H, M), lambda i: (0, 0)),
            pl.BlockSpec((1, H), lambda i: (0, 0)),
            pl.BlockSpec((1, M), lambda i: (0, 0)),
        ],
        out_specs=[
            pl.BlockSpec((R, M), lambda i: (i, 0)),
            pl.BlockSpec((R, H), lambda i: (i, 0)),
        ],
        out_shape=[
            jax.ShapeDtypeStruct((N, M), jnp.float32),
            jax.ShapeDtypeStruct((N, H), jnp.float32),
        ],
        scratch_shapes=[pltpu.VMEM((R, 1), jnp.int32)],
    )(x, enc_t, dec_t, bpre, benc)

    return z_n.reshape(B, D, L, M), x_tgt.reshape(B, D, L, H)


# speedup vs baseline: 1.1118x; 1.0023x over previous
import functools

import jax
import jax.numpy as jnp
from jax.experimental import pallas as pl
from jax.experimental.pallas import tpu as pltpu

_TOPK = 64


def _sae_body(x_ref, enc_ref, dec_ref, bpre_ref, benc_ref, zn_ref, xt_ref,
              acc_ref, *, topk):
    x = x_ref[...]
    xb = x - bpre_ref[...]
    logits = jax.lax.dot_general(
        xb, enc_ref[...], (((1,), (1,)), ((), ())),
        preferred_element_type=jnp.float32,
        precision=jax.lax.Precision.DEFAULT)
    z = jnp.maximum(logits + benc_ref[...], 0.0)

    topkf = jnp.float32(topk)
    acc = jnp.zeros((z.shape[0], 1), jnp.int32)
    cnta = jnp.full((z.shape[0], 1), jnp.float32(z.shape[1]))
    for b in range(30, 4, -1):
        cand = acc | (1 << b)
        candf = jax.lax.bitcast_convert_type(cand, jnp.float32)
        cnt = jnp.sum(jnp.where(z >= candf, 1.0, 0.0), axis=1, keepdims=True)
        take = cnt >= topkf
        acc = jnp.where(take, cand, acc)
        cnta = jnp.where(take, cnt, cnta)
    acc_ref[...] = acc

    @pl.when(jnp.any(cnta != topkf))
    def _refine():
        a = acc_ref[...]
        for b in range(4, -1, -1):
            cand = a | (1 << b)
            candf = jax.lax.bitcast_convert_type(cand, jnp.float32)
            cnt = jnp.sum(jnp.where(z >= candf, 1.0, 0.0),
                          axis=1, keepdims=True)
            a = jnp.where(cnt >= topkf, cand, a)
        acc_ref[...] = a

    thr = jax.lax.bitcast_convert_type(acc_ref[...], jnp.float32)

    zs = jnp.where(z >= thr, z, 0.0)
    zn_ref[...] = zs
    xt = jax.lax.dot_general(
        zs, dec_ref[...], (((1,), (1,)), ((), ())),
        preferred_element_type=jnp.float32,
        precision=jax.lax.Precision.DEFAULT)
    xt_ref[...] = xt + bpre_ref[...]


def kernel(zL, dictionary_enc, dictionary_dec, bias_pre, bias_enc):
    B, D, L, H = zL.shape
    M = dictionary_enc.shape[0]
    N = B * D * L
    R = 512 if N % 512 == 0 else N
    grid = N // R

    x = zL.reshape(N, H)
    enc_t = dictionary_enc
    dec_t = dictionary_dec
    bpre = bias_pre.reshape(1, H)
    benc = bias_enc.reshape(1, M)

    z_n, x_tgt = pl.pallas_call(
        functools.partial(_sae_body, topk=_TOPK),
        grid=(grid,),
        in_specs=[
            pl.BlockSpec((R, H), lambda i: (i, 0)),
            pl.BlockSpec((M, H), lambda i: (0, 0)),
            pl.BlockSpec((H, M), lambda i: (0, 0)),
            pl.BlockSpec((1, H), lambda i: (0, 0)),
            pl.BlockSpec((1, M), lambda i: (0, 0)),
        ],
        out_specs=[
            pl.BlockSpec((R, M), lambda i: (i, 0)),
            pl.BlockSpec((R, H), lambda i: (i, 0)),
        ],
        out_shape=[
            jax.ShapeDtypeStruct((N, M), jnp.float32),
            jax.ShapeDtypeStruct((N, H), jnp.float32),
        ],
        scratch_shapes=[pltpu.VMEM((R, 1), jnp.int32)],
    )(x, enc_t, dec_t, bpre, benc)

    return z_n.reshape(B, D, L, M), x_tgt.reshape(B, D, L, H)
